# baseline (speedup 1.0000x reference)
import functools

import jax
import jax.numpy as jnp
from jax import lax
from jax.experimental import pallas as pl
from jax.experimental.pallas import tpu as pltpu
from jax.experimental.pallas import tpu_sc as plsc

N_OUT = 64
ROWS = 64
COLS = 8192
LANES = 16
CHUNK = 64
N_CHUNKS = COLS // CHUNK
N_WORKERS = 32
ROWS_PER_WORKER = ROWS // N_WORKERS


def _vsort(v):
    k, _ = plsc.sort_key_val(v, v)
    return k


def _rev(v):
    return lax.rev(v, dimensions=(0,))


def _merge2(a, b):
    rb = _rev(b)
    lo = jnp.minimum(a, rb)
    hi = jnp.maximum(a, rb)
    return _vsort(lo), _vsort(hi)


def _merge4(a0, a1, b0, b1):
    rb0, rb1 = _rev(b1), _rev(b0)
    lo0 = jnp.minimum(a0, rb0)
    lo1 = jnp.minimum(a1, rb1)
    hi0 = jnp.maximum(a0, rb0)
    hi1 = jnp.maximum(a1, rb1)
    l0 = jnp.minimum(lo0, lo1)
    l1 = jnp.maximum(lo0, lo1)
    h0 = jnp.minimum(hi0, hi1)
    h1 = jnp.maximum(hi0, hi1)
    return _vsort(l0), _vsort(l1), _vsort(h0), _vsort(h1)


def _sort64(c0, c1, c2, c3):
    a0, a1 = _merge2(_vsort(c0), _vsort(c1))
    b0, b1 = _merge2(_vsort(c2), _vsort(c3))
    return _merge4(a0, a1, b0, b1)


def _keep_top64(r, c):
    r0, r1, r2, r3 = r
    c0, c1, c2, c3 = c
    rc0, rc1, rc2, rc3 = _rev(c3), _rev(c2), _rev(c1), _rev(c0)
    hi0 = jnp.maximum(r0, rc0)
    hi1 = jnp.maximum(r1, rc1)
    hi2 = jnp.maximum(r2, rc2)
    hi3 = jnp.maximum(r3, rc3)
    l0 = jnp.minimum(hi0, hi2)
    l1 = jnp.minimum(hi1, hi3)
    u0 = jnp.maximum(hi0, hi2)
    u1 = jnp.maximum(hi1, hi3)
    p0 = jnp.minimum(l0, l1)
    p1 = jnp.maximum(l0, l1)
    q0 = jnp.minimum(u0, u1)
    q1 = jnp.maximum(u0, u1)
    return _vsort(p0), _vsort(p1), _vsort(q0), _vsort(q1)


def _row_top64(row_v):
    def load4(base):
        return [row_v[pl.ds(base + k * LANES, LANES)] for k in range(4)]

    r = _sort64(*load4(0))
    rmin = lax.reduce_min(r[0], axes=(0,))

    def body(i, carry):
        r0, r1, r2, r3, rmin = carry
        c = load4(i * CHUNK)
        cmax = lax.reduce_max(
            jnp.maximum(jnp.maximum(c[0], c[1]), jnp.maximum(c[2], c[3])),
            axes=(0,),
        )

        def do_merge(_):
            nr = _keep_top64((r0, r1, r2, r3), _sort64(*c))
            return nr + (lax.reduce_min(nr[0], axes=(0,)),)

        def skip(_):
            return (r0, r1, r2, r3, rmin)

        return lax.cond(cmax > rmin, do_merge, skip, None)

    carry = lax.fori_loop(1, N_CHUNKS, body, r + (rmin,))
    return carry[:4]


@functools.partial(
    pl.kernel,
    out_type=jax.ShapeDtypeStruct((ROWS, N_OUT), jnp.float32),
    mesh=plsc.VectorSubcoreMesh(core_axis_name="c", subcore_axis_name="s"),
    scratch_types=[
        pltpu.VMEM((COLS,), jnp.float32),
        pltpu.VMEM((N_OUT,), jnp.float32),
    ],
    compiler_params=pltpu.CompilerParams(needs_layout_passes=False),
)
def _sc_topn(x_hbm, o_hbm, row_v, out_v):
    wid = lax.axis_index("s") * 2 + lax.axis_index("c")
    for rr in range(ROWS_PER_WORKER):
        row = wid * ROWS_PER_WORKER + rr
        pltpu.sync_copy(x_hbm.at[row], row_v)
        r0, r1, r2, r3 = _row_top64(row_v)
        out_v[pl.ds(0, LANES)] = _rev(r3)
        out_v[pl.ds(16, LANES)] = _rev(r2)
        out_v[pl.ds(32, LANES)] = _rev(r1)
        out_v[pl.ds(48, LANES)] = _rev(r0)
        pltpu.sync_copy(out_v, o_hbm.at[row])


def kernel(inputs):
    return _sc_topn(inputs)


# speedup vs baseline: 1.7078x; 1.7078x over previous
import functools

import jax
import jax.numpy as jnp
from jax import lax
from jax.experimental import pallas as pl
from jax.experimental.pallas import tpu as pltpu
from jax.experimental.pallas import tpu_sc as plsc

N_OUT = 64
ROWS = 64
COLS = 8192
LANES = 16
CHUNK = 64
GROUP = 4 * CHUNK
N_GROUPS = COLS // GROUP
N_WORKERS = 32
ROWS_PER_WORKER = ROWS // N_WORKERS


def _vsort(v):
    k, _ = plsc.sort_key_val(v, v)
    return k


def _rev(v):
    return lax.rev(v, dimensions=(0,))


def _merge2(a, b):
    rb = _rev(b)
    lo = jnp.minimum(a, rb)
    hi = jnp.maximum(a, rb)
    return _vsort(lo), _vsort(hi)


def _merge4(a0, a1, b0, b1):
    rb0, rb1 = _rev(b1), _rev(b0)
    lo0 = jnp.minimum(a0, rb0)
    lo1 = jnp.minimum(a1, rb1)
    hi0 = jnp.maximum(a0, rb0)
    hi1 = jnp.maximum(a1, rb1)
    l0 = jnp.minimum(lo0, lo1)
    l1 = jnp.maximum(lo0, lo1)
    h0 = jnp.minimum(hi0, hi1)
    h1 = jnp.maximum(hi0, hi1)
    return _vsort(l0), _vsort(l1), _vsort(h0), _vsort(h1)


def _sort64(c0, c1, c2, c3):
    a0, a1 = _merge2(_vsort(c0), _vsort(c1))
    b0, b1 = _merge2(_vsort(c2), _vsort(c3))
    return _merge4(a0, a1, b0, b1)


def _keep_top64(r, c):
    r0, r1, r2, r3 = r
    c0, c1, c2, c3 = c
    rc0, rc1, rc2, rc3 = _rev(c3), _rev(c2), _rev(c1), _rev(c0)
    hi0 = jnp.maximum(r0, rc0)
    hi1 = jnp.maximum(r1, rc1)
    hi2 = jnp.maximum(r2, rc2)
    hi3 = jnp.maximum(r3, rc3)
    l0 = jnp.minimum(hi0, hi2)
    l1 = jnp.minimum(hi1, hi3)
    u0 = jnp.maximum(hi0, hi2)
    u1 = jnp.maximum(hi1, hi3)
    p0 = jnp.minimum(l0, l1)
    p1 = jnp.maximum(l0, l1)
    q0 = jnp.minimum(u0, u1)
    q1 = jnp.maximum(u0, u1)
    return _vsort(p0), _vsort(p1), _vsort(q0), _vsort(q1)


def _group_top64(row_v, base):
    def sorted_chunk(cbase):
        c = [row_v[pl.ds(cbase + k * LANES, LANES)] for k in range(4)]
        return _sort64(*c)

    s0 = sorted_chunk(base)
    s1 = sorted_chunk(base + CHUNK)
    s2 = sorted_chunk(base + 2 * CHUNK)
    s3 = sorted_chunk(base + 3 * CHUNK)
    return _keep_top64(_keep_top64(s0, s1), _keep_top64(s2, s3))


@functools.partial(
    pl.kernel,
    out_type=jax.ShapeDtypeStruct((ROWS, N_OUT), jnp.float32),
    mesh=plsc.VectorSubcoreMesh(core_axis_name="c", subcore_axis_name="s"),
    scratch_types=[
        pltpu.VMEM((COLS,), jnp.float32),
        pltpu.VMEM((COLS,), jnp.float32),
        pltpu.VMEM((N_OUT,), jnp.float32),
        pltpu.VMEM((N_OUT,), jnp.float32),
        pltpu.SemaphoreType.DMA,
        pltpu.SemaphoreType.DMA,
    ],
    compiler_params=pltpu.CompilerParams(needs_layout_passes=False),
)
def _sc_topn(x_hbm, o_hbm, rowa_v, rowb_v, outa_v, outb_v, sem_a, sem_b):
    wid = lax.axis_index("s") * 2 + lax.axis_index("c")
    row_a = wid * ROWS_PER_WORKER
    row_b = row_a + 1
    cp_a = pltpu.async_copy(x_hbm.at[row_a], rowa_v, sem_a)
    cp_b = pltpu.async_copy(x_hbm.at[row_b], rowb_v, sem_b)
    cp_a.wait()
    cp_b.wait()

    neg_inf = jnp.full((LANES,), -jnp.inf, jnp.float32)
    r_init = (neg_inf, neg_inf, neg_inf, neg_inf)

    def body(i, carry):
        ra, rb = carry[:4], carry[4:]
        base = i * GROUP
        ra = _keep_top64(ra, _group_top64(rowa_v, base))
        rb = _keep_top64(rb, _group_top64(rowb_v, base))
        return ra + rb

    carry = lax.fori_loop(0, N_GROUPS, body, r_init + r_init)
    ra, rb = carry[:4], carry[4:]

    for out_v, r in ((outa_v, ra), (outb_v, rb)):
        out_v[pl.ds(0, LANES)] = _rev(r[3])
        out_v[pl.ds(16, LANES)] = _rev(r[2])
        out_v[pl.ds(32, LANES)] = _rev(r[1])
        out_v[pl.ds(48, LANES)] = _rev(r[0])
    pltpu.sync_copy(outa_v, o_hbm.at[row_a])
    pltpu.sync_copy(outb_v, o_hbm.at[row_b])


def kernel(inputs):
    return _sc_topn(inputs)
